# baseline (speedup 1.0000x reference)
import functools

import jax
import jax.numpy as jnp
from jax.experimental import pallas as pl
from jax.experimental.pallas import tpu as pltpu


def _moe_body(x_ref, W1_ref, b1_ref, W2_ref, b2_ref,
              Wg1_ref, bg1_ref, Wg2_ref, bg2_ref, Wc_ref, bc_ref,
              out_ref, scores_ref, W2c_ref, bc2_ref, W1v_ref, Wg1v_ref,
              w1_sem, wg1_sem, *, n_experts):
    E = n_experts
    i = pl.program_id(0)
    w1_copy = pltpu.make_async_copy(W1_ref, W1v_ref, w1_sem)
    wg1_copy = pltpu.make_async_copy(Wg1_ref, Wg1v_ref, wg1_sem)

    @pl.when(i == 0)
    def _start_prefetch():
        w1_copy.start()
        wg1_copy.start()

    @pl.when(i < E)
    def _fold():
        Wc = Wc_ref[...]
        prod = jnp.dot(W2_ref[0], Wc, preferred_element_type=jnp.float32)
        W2c_ref[pl.ds(i, 1)] = prod.astype(jnp.bfloat16)[None]

        @pl.when(i == 0)
        def _fold_bias():
            r = jnp.dot(b2_ref[...].astype(jnp.bfloat16), Wc,
                        preferred_element_type=jnp.float32) + bc_ref[...]
            bc2_ref[...] = r[:, None, :]

    @pl.when(i == E)
    def _wait_prefetch():
        w1_copy.wait()
        wg1_copy.wait()

    @pl.when(i >= E)
    def _tokens():
        W1_ref = W1v_ref
        Wg1_ref = Wg1v_ref
        xblk = x_ref[...]

        for e in range(E):
            h = jnp.dot(xblk, W1_ref[e], preferred_element_type=jnp.float32)
            h = jnp.maximum(h + b1_ref[e:e + 1, :], 0.0)
            o = jnp.dot(h.astype(jnp.bfloat16), W2c_ref[e],
                        preferred_element_type=jnp.float32) + bc2_ref[e]
            out_ref[:, e, :] = o

        g = jnp.dot(xblk, Wg1_ref[...], preferred_element_type=jnp.float32)
        g = jnp.maximum(g + bg1_ref[...], 0.0)
        logits = jnp.dot(g.astype(jnp.bfloat16), Wg2_ref[...],
                         preferred_element_type=jnp.float32) + bg2_ref[...]
        m = jnp.max(logits, axis=1, keepdims=True)
        ex = jnp.exp(logits - m)
        scores_ref[...] = ex / jnp.sum(ex, axis=1, keepdims=True)


def kernel(x, W1, b1, W2, b2, Wg1, bg1, Wg2, bg2, Wc, bc):
    B, D = x.shape
    E, _, H = W1.shape
    C = Wc.shape[1]
    BT = 256 if B % 256 == 0 else B
    T = B // BT

    bf = jnp.bfloat16
    xb = x.astype(bf)
    W1b, W2b = W1.astype(bf), W2.astype(bf)
    Wg1b, Wg2b, Wcb = Wg1.astype(bf), Wg2.astype(bf), Wc.astype(bf)
    bg1_2 = bg1.reshape(1, D)
    bg2_2 = bg2.reshape(1, E)
    bc_2 = bc.reshape(1, C)

    tok = lambda i: jnp.maximum(i - E, 0)
    whole = lambda *dims: pl.BlockSpec(dims, lambda i: (0,) * len(dims))
    out, scores = pl.pallas_call(
        functools.partial(_moe_body, n_experts=E),
        grid=(E + T,),
        in_specs=[
            pl.BlockSpec((BT, D), lambda i: (tok(i), 0)),
            pl.BlockSpec(memory_space=pl.ANY),
            whole(E, H),
            pl.BlockSpec((1, H, H),
                         lambda i: (jnp.minimum(i, E - 1), 0, 0)),
            whole(E, H),
            pl.BlockSpec(memory_space=pl.ANY),
            whole(1, D),
            whole(D, E),
            whole(1, E),
            whole(H, C),
            whole(1, C),
        ],
        out_specs=[
            pl.BlockSpec((BT, E, C), lambda i: (tok(i), 0, 0)),
            pl.BlockSpec((BT, E), lambda i: (tok(i), 0)),
        ],
        out_shape=[
            jax.ShapeDtypeStruct((B, E, C), jnp.float32),
            jax.ShapeDtypeStruct((B, E), jnp.float32),
        ],
        scratch_shapes=[
            pltpu.VMEM((E, H, C), bf),
            pltpu.VMEM((E, 1, C), jnp.float32),
            pltpu.VMEM((E, D, H), bf),
            pltpu.VMEM((D, D), bf),
            pltpu.SemaphoreType.DMA,
            pltpu.SemaphoreType.DMA,
        ],
        compiler_params=pltpu.CompilerParams(
            vmem_limit_bytes=63 * 1024 * 1024,
        ),
    )(xb, W1b, b1, W2b, b2, Wg1b, bg1_2, Wg2b, bg2_2, Wcb, bc_2)
    return (out, scores)


# speedup vs baseline: 1.0064x; 1.0064x over previous
import functools

import jax
import jax.numpy as jnp
from jax.experimental import pallas as pl
from jax.experimental.pallas import tpu as pltpu


def _moe_body(x_ref, W1_ref, b1_ref, W2_ref, b2_ref,
              Wg1_ref, bg1_ref, Wg2_ref, bg2_ref, Wc_ref, bc_ref,
              out_ref, scores_ref, W2c_ref, bc2_ref, W1v_ref, Wg1v_ref,
              w1_sem, wg1_sem, *, n_experts, fold_per_step):
    E = n_experts
    EF = fold_per_step
    NF = E // EF
    i = pl.program_id(0)
    w1_copy = pltpu.make_async_copy(W1_ref, W1v_ref, w1_sem)
    wg1_copy = pltpu.make_async_copy(Wg1_ref, Wg1v_ref, wg1_sem)

    @pl.when(i == 0)
    def _start_prefetch():
        w1_copy.start()
        wg1_copy.start()

    @pl.when(i < NF)
    def _fold():
        Wc = Wc_ref[...]
        for k in range(EF):
            prod = jnp.dot(W2_ref[k], Wc, preferred_element_type=jnp.float32)
            W2c_ref[pl.ds(i * EF + k, 1)] = prod.astype(jnp.bfloat16)[None]

        @pl.when(i == 0)
        def _fold_bias():
            r = jnp.dot(b2_ref[...].astype(jnp.bfloat16), Wc,
                        preferred_element_type=jnp.float32) + bc_ref[...]
            bc2_ref[...] = r[:, None, :]

    @pl.when(i == NF)
    def _wait_prefetch():
        w1_copy.wait()
        wg1_copy.wait()

    @pl.when(i >= NF)
    def _tokens():
        W1_ref = W1v_ref
        Wg1_ref = Wg1v_ref
        xblk = x_ref[...]

        g = jnp.dot(xblk, Wg1_ref[...], preferred_element_type=jnp.float32)
        g = jnp.maximum(g + bg1_ref[...], 0.0)
        logits = jnp.dot(g.astype(jnp.bfloat16), Wg2_ref[...],
                         preferred_element_type=jnp.float32) + bg2_ref[...]
        m = jnp.max(logits, axis=1, keepdims=True)
        ex = jnp.exp(logits - m)
        scores_ref[...] = ex / jnp.sum(ex, axis=1, keepdims=True)

        for e in range(E):
            h = jnp.dot(xblk, W1_ref[e], preferred_element_type=jnp.float32)
            h = jnp.maximum(h + b1_ref[e:e + 1, :], 0.0)
            o = jnp.dot(h.astype(jnp.bfloat16), W2c_ref[e],
                        preferred_element_type=jnp.float32) + bc2_ref[e]
            out_ref[:, e, :] = o


def kernel(x, W1, b1, W2, b2, Wg1, bg1, Wg2, bg2, Wc, bc):
    B, D = x.shape
    E, _, H = W1.shape
    C = Wc.shape[1]
    BT = 256 if B % 256 == 0 else B
    T = B // BT
    EF = 1
    NF = E // EF

    bf = jnp.bfloat16
    xb = x.astype(bf)
    W1b, W2b = W1.astype(bf), W2.astype(bf)
    Wg1b, Wg2b, Wcb = Wg1.astype(bf), Wg2.astype(bf), Wc.astype(bf)
    bg1_2 = bg1.reshape(1, D)
    bg2_2 = bg2.reshape(1, E)
    bc_2 = bc.reshape(1, C)

    tok = lambda i: jnp.maximum(i - NF, 0)
    whole = lambda *dims: pl.BlockSpec(dims, lambda i: (0,) * len(dims))
    out, scores = pl.pallas_call(
        functools.partial(_moe_body, n_experts=E, fold_per_step=EF),
        grid=(NF + T,),
        in_specs=[
            pl.BlockSpec((BT, D), lambda i: (tok(i), 0)),
            pl.BlockSpec(memory_space=pl.ANY),
            whole(E, H),
            pl.BlockSpec((EF, H, H),
                         lambda i: (jnp.minimum(i, NF - 1), 0, 0)),
            whole(E, H),
            pl.BlockSpec(memory_space=pl.ANY),
            whole(1, D),
            whole(D, E),
            whole(1, E),
            whole(H, C),
            whole(1, C),
        ],
        out_specs=[
            pl.BlockSpec((BT, E, C), lambda i: (tok(i), 0, 0)),
            pl.BlockSpec((BT, E), lambda i: (tok(i), 0)),
        ],
        out_shape=[
            jax.ShapeDtypeStruct((B, E, C), jnp.float32),
            jax.ShapeDtypeStruct((B, E), jnp.float32),
        ],
        scratch_shapes=[
            pltpu.VMEM((E, H, C), bf),
            pltpu.VMEM((E, 1, C), jnp.float32),
            pltpu.VMEM((E, D, H), bf),
            pltpu.VMEM((D, D), bf),
            pltpu.SemaphoreType.DMA,
            pltpu.SemaphoreType.DMA,
        ],
        compiler_params=pltpu.CompilerParams(
            vmem_limit_bytes=63 * 1024 * 1024,
        ),
    )(xb, W1b, b1, W2b, b2, Wg1b, bg1_2, Wg2b, bg2_2, Wcb, bc_2)
    return (out, scores)


# speedup vs baseline: 1.0993x; 1.0922x over previous
import functools

import jax
import jax.numpy as jnp
from jax.experimental import pallas as pl
from jax.experimental.pallas import tpu as pltpu


def _moe_body(x_ref, W1_ref, b1_ref, W2_ref, b2_ref,
              Wg1_ref, bg1_ref, Wg2_ref, bg2_ref, Wc_ref, bc_ref,
              out_ref, scores_ref, W2c_ref, bc2_ref, W1v_ref, Wg1v_ref,
              w1_sem, wg1_sem, *, n_experts):
    E = n_experts
    HH = W2_ref.shape[1]
    NF = 2 * E
    i = pl.program_id(0)
    w1_copy = pltpu.make_async_copy(W1_ref, W1v_ref, w1_sem)
    wg1_copy = pltpu.make_async_copy(Wg1_ref, Wg1v_ref, wg1_sem)

    @pl.when(i == 0)
    def _start_prefetch():
        w1_copy.start()
        wg1_copy.start()

    @pl.when(i < NF)
    def _fold():
        Wc = Wc_ref[...]
        prod = jnp.dot(W2_ref[0].astype(jnp.bfloat16), Wc,
                       preferred_element_type=jnp.float32)
        W2c_ref[pl.ds(i // 2, 1), pl.ds((i % 2) * HH, HH), :] = (
            prod.astype(jnp.bfloat16)[None])

        @pl.when(i == 0)
        def _fold_bias():
            r = jnp.dot(b2_ref[...].astype(jnp.bfloat16), Wc,
                        preferred_element_type=jnp.float32) + bc_ref[...]
            bc2_ref[...] = r[:, None, :]

    @pl.when(i == NF)
    def _wait_prefetch():
        w1_copy.wait()
        wg1_copy.wait()

    @pl.when(i >= NF)
    def _tokens():
        W1_ref = W1v_ref
        Wg1_ref = Wg1v_ref
        xblk = x_ref[...].astype(jnp.bfloat16)

        g = jnp.dot(xblk, Wg1_ref[...], preferred_element_type=jnp.float32)
        g = jnp.maximum(g + bg1_ref[...], 0.0)
        logits = jnp.dot(g.astype(jnp.bfloat16), Wg2_ref[...],
                         preferred_element_type=jnp.float32) + bg2_ref[...]
        m = jnp.max(logits, axis=1, keepdims=True)
        ex = jnp.exp(logits - m)
        scores_ref[...] = ex / jnp.sum(ex, axis=1, keepdims=True)

        for e in range(E):
            h = jnp.dot(xblk, W1_ref[e], preferred_element_type=jnp.float32)
            h = jnp.maximum(h + b1_ref[e:e + 1, :], 0.0)
            o = jnp.dot(h.astype(jnp.bfloat16), W2c_ref[e],
                        preferred_element_type=jnp.float32) + bc2_ref[e]
            out_ref[:, e, :] = o


def kernel(x, W1, b1, W2, b2, Wg1, bg1, Wg2, bg2, Wc, bc):
    B, D = x.shape
    E, _, H = W1.shape
    C = Wc.shape[1]
    BT = 256 if B % 256 == 0 else B
    T = B // BT
    NF = 2 * E

    bf = jnp.bfloat16
    W1b = W1.astype(bf)
    Wg1b, Wg2b, Wcb = Wg1.astype(bf), Wg2.astype(bf), Wc.astype(bf)
    bg1_2 = bg1.reshape(1, D)
    bg2_2 = bg2.reshape(1, E)
    bc_2 = bc.reshape(1, C)

    tok = lambda i: jnp.maximum(i - NF, 0)
    w2m = lambda i: jnp.minimum(i, NF - 1)
    whole = lambda *dims: pl.BlockSpec(dims, lambda i: (0,) * len(dims))
    out, scores = pl.pallas_call(
        functools.partial(_moe_body, n_experts=E),
        grid=(NF + T,),
        in_specs=[
            pl.BlockSpec((BT, D), lambda i: (tok(i), 0)),
            pl.BlockSpec(memory_space=pl.ANY),
            whole(E, H),
            pl.BlockSpec((1, H // 2, H),
                         lambda i: (w2m(i) // 2, w2m(i) % 2, 0)),
            whole(E, H),
            pl.BlockSpec(memory_space=pl.ANY),
            whole(1, D),
            whole(D, E),
            whole(1, E),
            whole(H, C),
            whole(1, C),
        ],
        out_specs=[
            pl.BlockSpec((BT, E, C), lambda i: (tok(i), 0, 0)),
            pl.BlockSpec((BT, E), lambda i: (tok(i), 0)),
        ],
        out_shape=[
            jax.ShapeDtypeStruct((B, E, C), jnp.float32),
            jax.ShapeDtypeStruct((B, E), jnp.float32),
        ],
        scratch_shapes=[
            pltpu.VMEM((E, H, C), bf),
            pltpu.VMEM((E, 1, C), jnp.float32),
            pltpu.VMEM((E, D, H), bf),
            pltpu.VMEM((D, D), bf),
            pltpu.SemaphoreType.DMA,
            pltpu.SemaphoreType.DMA,
        ],
        compiler_params=pltpu.CompilerParams(
            vmem_limit_bytes=63 * 1024 * 1024,
        ),
    )(x, W1b, b1, W2, b2, Wg1b, bg1_2, Wg2b, bg2_2, Wcb, bc_2)
    return (out, scores)
